# baseline (speedup 1.0000x reference)
import functools

import jax
import jax.numpy as jnp
from jax import lax
from jax.experimental import pallas as pl
from jax.experimental.pallas import tpu as pltpu
from jax.experimental.pallas import tpu_sc as plsc

_N = 16384
_NC = 2
_NS = 16
_NW = _NC * _NS
_L = 16
_BPW = _N // _NW
_QG = 128
_NQ = _BPW // _QG


def _body(ti_hbm, labels_hbm, pt_hbm, out_hbm,
          ti_v, a_v, b_v, g_v, lab_v, out_v, sem):
    a_hbm = pt_hbm.at[0]
    b_hbm = pt_hbm.at[1]
    g_hbm = pt_hbm.at[2]
    cid = lax.axis_index("c")
    sid = lax.axis_index("s")
    wid = sid * _NC + cid
    base = wid * _BPW
    lbase = pl.multiple_of(lax.max(base - _L, 0), _L)

    for q in range(_NQ):
        pltpu.sync_copy(ti_hbm.at[pl.ds(base + q * _QG, _QG)], ti_v.at[q])
    pltpu.sync_copy(labels_hbm.at[pl.ds(lbase, _BPW + _L)], lab_v)

    copies = [
        pltpu.async_copy(tab.at[ti_v.at[q]], dst.at[pl.ds(q * _QG, _QG)], sem)
        for tab, dst in ((a_hbm, a_v), (b_hbm, b_v), (g_hbm, g_v))
        for q in range(_NQ)
    ]
    for cp in copies:
        cp.wait()

    lane = lax.iota(jnp.int32, _L)

    def _chunk(j, carry):
        off = pl.multiple_of(j * _L, _L)
        iv = lane + (base + off)
        imp = jnp.maximum(iv - 1, 0) - lbase
        impp = jnp.maximum(iv - 2, 0) - lbase
        mp = plsc.load_gather(lab_v, [imp])
        mpp = plsc.load_gather(lab_v, [impp])
        sl = pl.ds(off, _L)
        out_v[sl] = a_v[sl] * mp + b_v[sl] * mpp + g_v[sl]
        return carry

    lax.fori_loop(0, _BPW // _L, _chunk, 0)

    pltpu.sync_copy(out_v, out_hbm.at[pl.ds(base, _BPW)])


@functools.partial(
    pl.kernel,
    mesh=plsc.VectorSubcoreMesh(core_axis_name="c", subcore_axis_name="s"),
    out_type=jax.ShapeDtypeStruct((_N,), jnp.float32),
    compiler_params=pltpu.CompilerParams(
        needs_layout_passes=False, use_tc_tiling_on_sc=False,
        disable_bounds_checks=True, disable_semaphore_checks=True,
        skip_device_barrier=True),
    scratch_types=[
        pltpu.VMEM((_NQ, _QG), jnp.int32),
        pltpu.VMEM((_BPW,), jnp.float32),
        pltpu.VMEM((_BPW,), jnp.float32),
        pltpu.VMEM((_BPW,), jnp.float32),
        pltpu.VMEM((_BPW + _L,), jnp.float32),
        pltpu.VMEM((_BPW,), jnp.float32),
        pltpu.SemaphoreType.DMA,
    ],
)
def _sc_predict(ti_hbm, labels_hbm, pt_hbm, out_hbm, *scratch):
    _body(ti_hbm, labels_hbm, pt_hbm, out_hbm, *scratch)


def kernel(train_indices, M_prev, M_prev_prev, labels, params):
    del M_prev, M_prev_prev
    pt = jnp.stack([params[:, 0], params[:, 1], params[:, 2]])
    return _sc_predict(train_indices.astype(jnp.int32), labels, pt)


# speedup vs baseline: 1.7529x; 1.7529x over previous
import functools

import jax
import jax.numpy as jnp
from jax import lax
from jax.experimental import pallas as pl
from jax.experimental.pallas import tpu as pltpu
from jax.experimental.pallas import tpu_sc as plsc

_N = 16384
_NC = 2
_NS = 16
_NW = _NC * _NS
_L = 16
_BPW = _N // _NW
_QG = 128
_NQ = _BPW // _QG


def _body(ti_hbm, labels_hbm, pt_hbm, out_hbm,
          ti_v, a_v, b_v, g_v, lab_v, out_v, sem, sem_i, sem_l):
    a_hbm = pt_hbm.at[0]
    b_hbm = pt_hbm.at[1]
    g_hbm = pt_hbm.at[2]
    cid = lax.axis_index("c")
    sid = lax.axis_index("s")
    wid = sid * _NC + cid
    base = wid * _BPW
    lbase = pl.multiple_of(lax.max(base - _L, 0), _L)

    icopies = [
        pltpu.async_copy(ti_hbm.at[pl.ds(base + q * _QG, _QG)], ti_v.at[q],
                         sem_i)
        for q in range(_NQ)
    ]
    lcopy = pltpu.async_copy(labels_hbm.at[pl.ds(lbase, _BPW + _L)], lab_v,
                             sem_l)
    gcopies = []
    for q in range(_NQ):
        icopies[q].wait()
        sl = pl.ds(q * _QG, _QG)
        for tab, dst in ((a_hbm, a_v), (b_hbm, b_v), (g_hbm, g_v)):
            gcopies.append(pltpu.async_copy(tab.at[ti_v.at[q]], dst.at[sl],
                                            sem))
    lcopy.wait()
    for cp in gcopies:
        cp.wait()

    lane = lax.iota(jnp.int32, _L)

    def _chunk(j, carry):
        off = pl.multiple_of(j * _L, _L)
        iv = lane + (base + off)
        imp = jnp.maximum(iv - 1, 0) - lbase
        impp = jnp.maximum(iv - 2, 0) - lbase
        mp = plsc.load_gather(lab_v, [imp])
        mpp = plsc.load_gather(lab_v, [impp])
        sl = pl.ds(off, _L)
        out_v[sl] = a_v[sl] * mp + b_v[sl] * mpp + g_v[sl]
        return carry

    lax.fori_loop(0, _BPW // _L, _chunk, 0)

    pltpu.sync_copy(out_v, out_hbm.at[pl.ds(base, _BPW)])


@functools.partial(
    pl.kernel,
    mesh=plsc.VectorSubcoreMesh(core_axis_name="c", subcore_axis_name="s"),
    out_type=jax.ShapeDtypeStruct((_N,), jnp.float32),
    compiler_params=pltpu.CompilerParams(
        needs_layout_passes=False, use_tc_tiling_on_sc=False,
        disable_bounds_checks=True, disable_semaphore_checks=True,
        skip_device_barrier=True),
    scratch_types=[
        pltpu.VMEM((_NQ, _QG), jnp.int32),
        pltpu.VMEM((_BPW,), jnp.float32),
        pltpu.VMEM((_BPW,), jnp.float32),
        pltpu.VMEM((_BPW,), jnp.float32),
        pltpu.VMEM((_BPW + _L,), jnp.float32),
        pltpu.VMEM((_BPW,), jnp.float32),
        pltpu.SemaphoreType.DMA,
        pltpu.SemaphoreType.DMA,
        pltpu.SemaphoreType.DMA,
    ],
)
def _sc_predict(ti_hbm, labels_hbm, pt_hbm, out_hbm, *scratch):
    _body(ti_hbm, labels_hbm, pt_hbm, out_hbm, *scratch)


def kernel(train_indices, M_prev, M_prev_prev, labels, params):
    del M_prev, M_prev_prev
    return _sc_predict(train_indices.astype(jnp.int32), labels, params.T)
